# baseline (speedup 1.0000x reference)
import functools

import jax
import jax.numpy as jnp
from jax import lax
from jax.experimental import pallas as pl
from jax.experimental.pallas import tpu as pltpu
from jax.experimental.pallas import tpu_sc as plsc

NUM_GRAPHS = 512
NPAD = 102400
R = 5120
ROWS_PER_TILE = NPAD // 16 // 128
ELEMS_PER_TILE = ROWS_PER_TILE * 128


def _tc_body(x_ref, pos_ref, W1_ref, Wp_ref, b1_ref, W2c_ref, WpW2_ref,
             e_ref, ft_ref):
    x = x_ref[...]
    z = jnp.dot(x, W1_ref[...], preferred_element_type=jnp.float32)
    z += lax.dot_general(pos_ref[...], Wp_ref[...],
                         (((0,), (0,)), ((), ())),
                         preferred_element_type=jnp.float32)
    z += b1_ref[...]

    sg = jnp.tanh(z * 0.5) * 0.5 + 0.5
    sz = z * sg
    e_row = lax.dot_general(W2c_ref[...], sz,
                            (((0,), (1,)), ((), ())),
                            preferred_element_type=jnp.float32)
    e_ref[...] = jnp.reshape(e_row, (e_ref.shape[0],))

    g = sg + sz * (1.0 - sg)
    ft_ref[...] = -lax.dot_general(WpW2_ref[...], g,
                                   (((1,), (1,)), ((), ())),
                                   preferred_element_type=jnp.float32)


def _sc_segsum(e2d, idx2d):
    mesh = plsc.VectorSubcoreMesh(core_axis_name="c", subcore_axis_name="s")

    @functools.partial(
        pl.kernel,
        out_type=jax.ShapeDtypeStruct((NUM_GRAPHS,), jnp.float32),
        mesh=mesh,
        scratch_types=[
            pltpu.VMEM((ELEMS_PER_TILE,), jnp.float32),
            pltpu.VMEM((ROWS_PER_TILE, 128), jnp.int32),
            pltpu.VMEM((1024,), jnp.float32),
            pltpu.VMEM_SHARED((1024,), jnp.float32),
        ],
    )
    def k(e_hbm, idx_hbm, out_hbm, e_v, idx_v, z_v, acc):
        c = lax.axis_index("c")
        s = lax.axis_index("s")

        @pl.when(c == 0)
        def _():
            base = pl.multiple_of(s * ELEMS_PER_TILE, 8)
            pltpu.sync_copy(e_hbm.at[pl.ds(base, ELEMS_PER_TILE)], e_v)
            pltpu.sync_copy(idx_hbm.at[s], idx_v)

            @pl.when(s == 0)
            def _():
                for i in range(1024 // 16):
                    z_v[pl.ds(16 * i, 16)] = jnp.zeros((16,), jnp.float32)
                pltpu.sync_copy(z_v, acc)

        plsc.subcore_barrier()

        @pl.when(c == 0)
        def _():
            for j in range(ROWS_PER_TILE):
                pltpu.sync_copy(e_v.at[pl.ds(j * 128, 128)],
                                acc.at[idx_v.at[j]], add=True)

        plsc.subcore_barrier()

        @pl.when((c == 0) & (s == 0))
        def _():
            pltpu.sync_copy(acc.at[pl.ds(0, NUM_GRAPHS)], out_hbm)

    return k(e2d, idx2d)


def kernel(x, pos, atomic_numbers, batch, W1, Wp, b1, W2):
    N, D = x.shape
    H = W1.shape[1]
    nblk = NPAD // R
    pos3 = jnp.pad(pos.T, ((0, 0), (0, NPAD - N)))

    e, ft = pl.pallas_call(
        _tc_body,
        grid=(nblk,),
        in_specs=[
            pl.BlockSpec((R, D), lambda i: (i, 0)),
            pl.BlockSpec((3, R), lambda i: (0, i)),
            pl.BlockSpec((D, H), lambda i: (0, 0)),
            pl.BlockSpec((3, H), lambda i: (0, 0)),
            pl.BlockSpec((1, H), lambda i: (0, 0)),
            pl.BlockSpec((H, 1), lambda i: (0, 0)),
            pl.BlockSpec((3, H), lambda i: (0, 0)),
        ],
        out_specs=[
            pl.BlockSpec((R,), lambda i: (i,)),
            pl.BlockSpec((3, R), lambda i: (0, i)),
        ],
        out_shape=[
            jax.ShapeDtypeStruct((NPAD,), jnp.float32),
            jax.ShapeDtypeStruct((3, NPAD), jnp.float32),
        ],
    )(x, pos3, W1, Wp.reshape(3, H), b1.reshape(1, H),
      W2.reshape(H, 1), Wp.reshape(3, H) * W2.reshape(1, H))

    idx = jnp.concatenate(
        [batch.astype(jnp.int32),
         jnp.full((NPAD - N,), NUM_GRAPHS, jnp.int32)]
    ).reshape(16, ROWS_PER_TILE, 128)
    pred_energy = _sc_segsum(e, idx)
    pred_forces = ft[:, :N].T
    return pred_energy, pred_forces


# speedup vs baseline: 1.0408x; 1.0408x over previous
import functools

import jax
import jax.numpy as jnp
from jax import lax
from jax.experimental import pallas as pl
from jax.experimental.pallas import tpu as pltpu
from jax.experimental.pallas import tpu_sc as plsc

NUM_GRAPHS = 512
NPAD = 102400
R = 10240
ROWS_PER_TILE = NPAD // 16 // 128
ELEMS_PER_TILE = ROWS_PER_TILE * 128


def _tc_body(x_ref, pos_ref, W1_ref, Wp_ref, b1_ref, W2c_ref, WpW2_ref,
             e_ref, ft_ref):
    x = x_ref[...]
    z = jnp.dot(x, W1_ref[...], preferred_element_type=jnp.float32)
    z += lax.dot_general(pos_ref[...], Wp_ref[...],
                         (((0,), (0,)), ((), ())),
                         preferred_element_type=jnp.float32)
    z += b1_ref[...]

    sg = jax.nn.sigmoid(z)
    sz = z * sg
    e_row = lax.dot_general(W2c_ref[...], sz,
                            (((0,), (1,)), ((), ())),
                            preferred_element_type=jnp.float32)
    e_ref[...] = jnp.reshape(e_row, (e_ref.shape[0],))

    g = sg + sz * (1.0 - sg)
    ft_ref[...] = -lax.dot_general(WpW2_ref[...], g,
                                   (((1,), (1,)), ((), ())),
                                   preferred_element_type=jnp.float32)


def _sc_segsum(e2d, idx2d):
    mesh = plsc.VectorSubcoreMesh(core_axis_name="c", subcore_axis_name="s")

    @functools.partial(
        pl.kernel,
        out_type=jax.ShapeDtypeStruct((NUM_GRAPHS,), jnp.float32),
        mesh=mesh,
        scratch_types=[
            pltpu.VMEM((ELEMS_PER_TILE,), jnp.float32),
            pltpu.VMEM((ROWS_PER_TILE, 128), jnp.int32),
            pltpu.VMEM((1024,), jnp.float32),
            pltpu.VMEM_SHARED((1024,), jnp.float32),
        ],
    )
    def k(e_hbm, idx_hbm, out_hbm, e_v, idx_v, z_v, acc):
        c = lax.axis_index("c")
        s = lax.axis_index("s")

        @pl.when(c == 0)
        def _():
            base = pl.multiple_of(s * ELEMS_PER_TILE, 8)
            pltpu.sync_copy(e_hbm.at[pl.ds(base, ELEMS_PER_TILE)], e_v)
            pltpu.sync_copy(idx_hbm.at[s], idx_v)

            @pl.when(s == 0)
            def _():
                for i in range(1024 // 16):
                    z_v[pl.ds(16 * i, 16)] = jnp.zeros((16,), jnp.float32)
                pltpu.sync_copy(z_v, acc)

        plsc.subcore_barrier()

        @pl.when(c == 0)
        def _():
            for j in range(ROWS_PER_TILE):
                pltpu.sync_copy(e_v.at[pl.ds(j * 128, 128)],
                                acc.at[idx_v.at[j]], add=True)

        plsc.subcore_barrier()

        @pl.when((c == 0) & (s == 0))
        def _():
            pltpu.sync_copy(acc.at[pl.ds(0, NUM_GRAPHS)], out_hbm)

    return k(e2d, idx2d)


def kernel(x, pos, atomic_numbers, batch, W1, Wp, b1, W2):
    N, D = x.shape
    H = W1.shape[1]
    nblk = NPAD // R
    pos3 = jnp.pad(pos.T, ((0, 0), (0, NPAD - N)))

    e, ft = pl.pallas_call(
        _tc_body,
        grid=(nblk,),
        in_specs=[
            pl.BlockSpec((R, D), lambda i: (i, 0)),
            pl.BlockSpec((3, R), lambda i: (0, i)),
            pl.BlockSpec((D, H), lambda i: (0, 0)),
            pl.BlockSpec((3, H), lambda i: (0, 0)),
            pl.BlockSpec((1, H), lambda i: (0, 0)),
            pl.BlockSpec((H, 1), lambda i: (0, 0)),
            pl.BlockSpec((3, H), lambda i: (0, 0)),
        ],
        out_specs=[
            pl.BlockSpec((R,), lambda i: (i,)),
            pl.BlockSpec((3, R), lambda i: (0, i)),
        ],
        out_shape=[
            jax.ShapeDtypeStruct((NPAD,), jnp.float32),
            jax.ShapeDtypeStruct((3, NPAD), jnp.float32),
        ],
    )(x, pos3, W1, Wp.reshape(3, H), b1.reshape(1, H),
      W2.reshape(H, 1), Wp.reshape(3, H) * W2.reshape(1, H))

    idx = jnp.concatenate(
        [batch.astype(jnp.int32),
         jnp.full((NPAD - N,), NUM_GRAPHS, jnp.int32)]
    ).reshape(16, ROWS_PER_TILE, 128)
    pred_energy = _sc_segsum(e, idx)
    pred_forces = ft[:, :N].T
    return pred_energy, pred_forces


# speedup vs baseline: 1.0501x; 1.0089x over previous
import functools

import jax
import jax.numpy as jnp
from jax import lax
from jax.experimental import pallas as pl
from jax.experimental.pallas import tpu as pltpu
from jax.experimental.pallas import tpu_sc as plsc

NUM_GRAPHS = 512
NPAD = 102400
R = 10240
ROWS_PER_TILE = NPAD // 16 // 128
ELEMS_PER_TILE = ROWS_PER_TILE * 128


def _tc_body(x_ref, pos_ref, W1_ref, Wp_ref, b1_ref, W2c_ref, W2r_ref,
             e_ref, ft_ref):
    x = x_ref[...]
    z = jnp.dot(x, W1_ref[...], preferred_element_type=jnp.float32)
    z += lax.dot_general(pos_ref[...], Wp_ref[...],
                         (((0,), (0,)), ((), ())),
                         preferred_element_type=jnp.float32)
    z += b1_ref[...]

    sg = jax.nn.sigmoid(z)
    sz = z * sg
    e_row = lax.dot_general(W2c_ref[...], sz,
                            (((0,), (1,)), ((), ())),
                            preferred_element_type=jnp.float32)
    e_ref[...] = jnp.reshape(e_row, (e_ref.shape[0],))

    g = (sg + sz * (1.0 - sg)) * W2r_ref[...]
    ft_ref[...] = -lax.dot_general(Wp_ref[...], g,
                                   (((1,), (1,)), ((), ())),
                                   preferred_element_type=jnp.float32)


def _sc_segsum(e2d, idx2d):
    mesh = plsc.VectorSubcoreMesh(core_axis_name="c", subcore_axis_name="s")

    @functools.partial(
        pl.kernel,
        out_type=jax.ShapeDtypeStruct((NUM_GRAPHS,), jnp.float32),
        mesh=mesh,
        scratch_types=[
            pltpu.VMEM((ELEMS_PER_TILE,), jnp.float32),
            pltpu.VMEM((ROWS_PER_TILE, 128), jnp.int32),
            pltpu.VMEM((1024,), jnp.float32),
            pltpu.VMEM_SHARED((1024,), jnp.float32),
            pltpu.SemaphoreType.DMA,
        ],
    )
    def k(e_hbm, idx_hbm, out_hbm, e_v, idx_v, z_v, acc, sem):
        c = lax.axis_index("c")
        s = lax.axis_index("s")

        @pl.when(c == 0)
        def _():
            base = pl.multiple_of(s * ELEMS_PER_TILE, 8)
            pltpu.sync_copy(e_hbm.at[pl.ds(base, ELEMS_PER_TILE)], e_v)
            pltpu.sync_copy(idx_hbm.at[s], idx_v)

            @pl.when(s == 0)
            def _():
                for i in range(1024 // 16):
                    z_v[pl.ds(16 * i, 16)] = jnp.zeros((16,), jnp.float32)
                pltpu.sync_copy(z_v, acc)

        plsc.subcore_barrier()

        @pl.when(c == 0)
        def _():
            handles = [
                pltpu.async_copy(e_v.at[pl.ds(j * 128, 128)],
                                 acc.at[idx_v.at[j]], sem, add=True)
                for j in range(ROWS_PER_TILE)]
            for h in handles:
                h.wait()

        plsc.subcore_barrier()

        @pl.when((c == 0) & (s == 0))
        def _():
            pltpu.sync_copy(acc.at[pl.ds(0, NUM_GRAPHS)], out_hbm)

    return k(e2d, idx2d)


def kernel(x, pos, atomic_numbers, batch, W1, Wp, b1, W2):
    N, D = x.shape
    H = W1.shape[1]
    nblk = NPAD // R
    pos3 = jnp.pad(pos.T, ((0, 0), (0, NPAD - N)))

    e, ft = pl.pallas_call(
        _tc_body,
        grid=(nblk,),
        in_specs=[
            pl.BlockSpec((R, D), lambda i: (i, 0)),
            pl.BlockSpec((3, R), lambda i: (0, i)),
            pl.BlockSpec((D, H), lambda i: (0, 0)),
            pl.BlockSpec((3, H), lambda i: (0, 0)),
            pl.BlockSpec((1, H), lambda i: (0, 0)),
            pl.BlockSpec((H, 1), lambda i: (0, 0)),
            pl.BlockSpec((1, H), lambda i: (0, 0)),
        ],
        out_specs=[
            pl.BlockSpec((R,), lambda i: (i,)),
            pl.BlockSpec((3, R), lambda i: (0, i)),
        ],
        out_shape=[
            jax.ShapeDtypeStruct((NPAD,), jnp.float32),
            jax.ShapeDtypeStruct((3, NPAD), jnp.float32),
        ],
    )(x, pos3, W1, Wp.reshape(3, H), b1.reshape(1, H),
      W2.reshape(H, 1), W2.reshape(1, H))

    idx = jnp.concatenate(
        [batch.astype(jnp.int32),
         jnp.full((NPAD - N,), NUM_GRAPHS, jnp.int32)]
    ).reshape(16, ROWS_PER_TILE, 128)
    pred_energy = _sc_segsum(e, idx)
    pred_forces = ft[:, :N].T
    return pred_energy, pred_forces
